# baseline (speedup 1.0000x reference)
import functools

import jax
import jax.numpy as jnp
from jax import lax
from jax.experimental import pallas as pl
from jax.experimental.pallas import tpu as pltpu
from jax.experimental.pallas import tpu_sc as plsc

NUM_EMB = 1000000
DIM = 64

NC = 2
NS = 16
NW = NC * NS

CHUNK = 128


def _gather_body(nchunk, idx_hbm, table_hbm, out_hbm, idx_v, rows_v,
                 gsem, wsem):
    wid = lax.axis_index("s") * NC + lax.axis_index("c")
    b_per_w = nchunk * CHUNK
    base = wid * b_per_w

    pltpu.sync_copy(idx_hbm.at[pl.ds(wid * nchunk, nchunk)], idx_v)

    def gather(g, slot):
        return pltpu.async_copy(table_hbm.at[idx_v.at[g]], rows_v.at[slot],
                                gsem)

    gather(0, 0)

    def loop_body(g, carry):
        slot = lax.rem(g, 2)

        @pl.when(g >= 1)
        def _():
            pltpu.make_async_copy(
                rows_v.at[slot], out_hbm.at[pl.ds(base, CHUNK)],
                wsem).wait()

        @pl.when(g + 1 < nchunk)
        def _():
            gather(g + 1, 1 - slot)

        pltpu.make_async_copy(table_hbm.at[idx_v.at[g]], rows_v.at[slot],
                              gsem).wait()

        pltpu.async_copy(rows_v.at[slot],
                         out_hbm.at[pl.ds(base + g * CHUNK, CHUNK)], wsem)
        return carry

    lax.fori_loop(0, nchunk, loop_body, 0)

    pltpu.make_async_copy(
        rows_v.at[0], out_hbm.at[pl.ds(base, CHUNK)], wsem).wait()


@functools.partial(jax.jit, static_argnames=("nchunk",))
def _run(x_flat2d, table, nchunk):
    mesh = plsc.VectorSubcoreMesh(core_axis_name="c", subcore_axis_name="s")
    kern = functools.partial(
        pl.kernel,
        out_type=jax.ShapeDtypeStruct((NW * nchunk * CHUNK, DIM),
                                      jnp.float32),
        mesh=mesh,
        scratch_types=[
            pltpu.VMEM((nchunk, CHUNK), jnp.int32),
            pltpu.VMEM((2, CHUNK, DIM), jnp.float32),
            pltpu.SemaphoreType.DMA,
            pltpu.SemaphoreType.DMA,
        ],
        compiler_params=pltpu.CompilerParams(use_tc_tiling_on_sc=False),
    )(functools.partial(_gather_body, nchunk))
    return kern(x_flat2d, table)


def kernel(x, table):
    b = x.shape[0] * x.shape[1]
    assert b % (NW * CHUNK) == 0
    nchunk = b // (NW * CHUNK)
    x_flat2d = x.reshape(NW * nchunk, CHUNK).astype(jnp.int32)
    out = _run(x_flat2d, table, nchunk)
    return out.reshape(x.shape[0], x.shape[1], DIM)


# speedup vs baseline: 1.0215x; 1.0215x over previous
import functools

import jax
import jax.numpy as jnp
from jax import lax
from jax.experimental import pallas as pl
from jax.experimental.pallas import tpu as pltpu
from jax.experimental.pallas import tpu_sc as plsc

NUM_EMB = 1000000
DIM = 64

NC = 2
NS = 16
NW = NC * NS

CHUNK = 512


def _gather_body(nchunk, idx_hbm, table_hbm, out_hbm, idx_v, rows_v,
                 gsem, wsem):
    wid = lax.axis_index("s") * NC + lax.axis_index("c")
    b_per_w = nchunk * CHUNK
    base = wid * b_per_w

    pltpu.sync_copy(idx_hbm.at[pl.ds(wid * nchunk, nchunk)], idx_v)

    def gather(g, slot):
        return pltpu.async_copy(table_hbm.at[idx_v.at[g]], rows_v.at[slot],
                                gsem)

    gather(0, 0)

    def loop_body(g, carry):
        slot = lax.rem(g, 2)

        @pl.when(g >= 1)
        def _():
            pltpu.make_async_copy(
                rows_v.at[slot], out_hbm.at[pl.ds(base, CHUNK)],
                wsem).wait()

        @pl.when(g + 1 < nchunk)
        def _():
            gather(g + 1, 1 - slot)

        pltpu.make_async_copy(table_hbm.at[idx_v.at[g]], rows_v.at[slot],
                              gsem).wait()

        pltpu.async_copy(rows_v.at[slot],
                         out_hbm.at[pl.ds(base + g * CHUNK, CHUNK)], wsem)
        return carry

    lax.fori_loop(0, nchunk, loop_body, 0)

    pltpu.make_async_copy(
        rows_v.at[0], out_hbm.at[pl.ds(base, CHUNK)], wsem).wait()


@functools.partial(jax.jit, static_argnames=("nchunk",))
def _run(x_flat2d, table, nchunk):
    mesh = plsc.VectorSubcoreMesh(core_axis_name="c", subcore_axis_name="s")
    kern = functools.partial(
        pl.kernel,
        out_type=jax.ShapeDtypeStruct((NW * nchunk * CHUNK, DIM),
                                      jnp.float32),
        mesh=mesh,
        scratch_types=[
            pltpu.VMEM((nchunk, CHUNK), jnp.int32),
            pltpu.VMEM((2, CHUNK, DIM), jnp.float32),
            pltpu.SemaphoreType.DMA,
            pltpu.SemaphoreType.DMA,
        ],
        compiler_params=pltpu.CompilerParams(use_tc_tiling_on_sc=False),
    )(functools.partial(_gather_body, nchunk))
    return kern(x_flat2d, table)


def kernel(x, table):
    b = x.shape[0] * x.shape[1]
    assert b % (NW * CHUNK) == 0
    nchunk = b // (NW * CHUNK)
    x_flat2d = x.reshape(NW * nchunk, CHUNK).astype(jnp.int32)
    out = _run(x_flat2d, table, nchunk)
    return out.reshape(x.shape[0], x.shape[1], DIM)
